# baseline (speedup 1.0000x reference)
import jax
import jax.numpy as jnp
from jax.experimental import pallas as pl

S = 2048
SPLIT = 512
S_BACK = S - SPLIT
D_MODEL = 1024
D_FF = 2048
E = 8
D_FF_E = 512

T_FFN = 256
T_MOE = 256


def _router_body(x_ref, w_ref, d_ref, comb_ref):
    l = jnp.dot(x_ref[...], w_ref[...], preferred_element_type=jnp.float32)
    l = l + d_ref[...]
    iota_e = jax.lax.broadcasted_iota(jnp.int32, l.shape, 1)
    m1 = jnp.max(l, axis=1, keepdims=True)
    a1 = jnp.min(jnp.where(l == m1, iota_e, E), axis=1, keepdims=True)
    masked = jnp.where(iota_e == a1, -jnp.inf, l)
    m2 = jnp.max(masked, axis=1, keepdims=True)
    a2 = jnp.min(jnp.where(masked == m2, iota_e, E), axis=1, keepdims=True)
    g1 = 1.0 / (1.0 + jnp.exp(m1 - m2))
    g0 = 1.0 - g1
    comb = jnp.where(iota_e == a1, g0, 0.0) + jnp.where(iota_e == a2, g1, 0.0)
    comb_ref[...] = comb


def _ffn_body(x_ref, w1_ref, b1_ref, w2_ref, b2_ref, y_ref):
    h = jnp.dot(x_ref[...], w1_ref[...], preferred_element_type=jnp.float32)
    h = jax.nn.gelu(h + b1_ref[...]).astype(jnp.bfloat16)
    y = jnp.dot(h, w2_ref[...], preferred_element_type=jnp.float32)
    y_ref[...] = y + b2_ref[...]


def _moe_body(x_ref, comb_ref, we1_ref, be1_ref, we2_ref, be2_ref, y_ref,
              out_ref):
    e = pl.program_id(1)
    h = jnp.dot(x_ref[...], we1_ref[0], preferred_element_type=jnp.float32)
    h = jax.nn.gelu(h + be1_ref[0]).astype(jnp.bfloat16)
    o = jnp.dot(h, we2_ref[0], preferred_element_type=jnp.float32)
    o = o + be2_ref[0]
    iota_e = jax.lax.broadcasted_iota(jnp.int32, comb_ref.shape, 1)
    g = jnp.sum(jnp.where(iota_e == e, comb_ref[...], 0.0), axis=1,
                keepdims=True)

    @pl.when(e == 0)
    def _():
        out_ref[...] = y_ref[...] + g * o

    @pl.when(e != 0)
    def _():
        out_ref[...] = out_ref[...] + g * o


def kernel(x, id, weight, delta, W1, b1, W2, b2, We1, be1, We2, be2):
    del id
    xf = x.reshape(S, D_MODEL)
    x_back = xf[SPLIT:]
    bf = jnp.bfloat16
    xf_b = xf.astype(bf)
    x_back_b = xf_b[SPLIT:]

    comb = pl.pallas_call(
        _router_body,
        grid=(S_BACK // T_MOE,),
        in_specs=[
            pl.BlockSpec((T_MOE, D_MODEL), lambda t: (t, 0)),
            pl.BlockSpec((D_MODEL, E), lambda t: (0, 0)),
            pl.BlockSpec((1, E), lambda t: (0, 0)),
        ],
        out_specs=pl.BlockSpec((T_MOE, E), lambda t: (t, 0)),
        out_shape=jax.ShapeDtypeStruct((S_BACK, E), jnp.float32),
    )(x_back, weight, delta.reshape(1, E))

    y = pl.pallas_call(
        _ffn_body,
        grid=(S // T_FFN,),
        in_specs=[
            pl.BlockSpec((T_FFN, D_MODEL), lambda t: (t, 0)),
            pl.BlockSpec((D_MODEL, D_FF), lambda t: (0, 0)),
            pl.BlockSpec((1, D_FF), lambda t: (0, 0)),
            pl.BlockSpec((D_FF, D_MODEL), lambda t: (0, 0)),
            pl.BlockSpec((1, D_MODEL), lambda t: (0, 0)),
        ],
        out_specs=pl.BlockSpec((T_FFN, D_MODEL), lambda t: (t, 0)),
        out_shape=jax.ShapeDtypeStruct((S, D_MODEL), jnp.float32),
    )(xf_b, W1.astype(bf), b1.reshape(1, D_FF), W2.astype(bf),
      b2.reshape(1, D_MODEL))

    out_back = pl.pallas_call(
        _moe_body,
        grid=(S_BACK // T_MOE, E),
        in_specs=[
            pl.BlockSpec((T_MOE, D_MODEL), lambda t, e: (t, 0)),
            pl.BlockSpec((T_MOE, E), lambda t, e: (t, 0)),
            pl.BlockSpec((1, D_MODEL, D_FF_E), lambda t, e: (e, 0, 0)),
            pl.BlockSpec((1, 1, D_FF_E), lambda t, e: (e, 0, 0)),
            pl.BlockSpec((1, D_FF_E, D_MODEL), lambda t, e: (e, 0, 0)),
            pl.BlockSpec((1, 1, D_MODEL), lambda t, e: (e, 0, 0)),
            pl.BlockSpec((T_MOE, D_MODEL), lambda t, e: (t, 0)),
        ],
        out_specs=pl.BlockSpec((T_MOE, D_MODEL), lambda t, e: (t, 0)),
        out_shape=jax.ShapeDtypeStruct((S_BACK, D_MODEL), jnp.float32),
    )(x_back_b, comb, We1.astype(bf), be1.reshape(E, 1, D_FF_E),
      We2.astype(bf), be2.reshape(E, 1, D_MODEL), y[SPLIT:])

    out = jnp.concatenate([y[:SPLIT], out_back], axis=0)
    return out.reshape(1, S, D_MODEL)


# speedup vs baseline: 1.4016x; 1.4016x over previous
import jax
import jax.numpy as jnp
from jax.experimental import pallas as pl

S = 2048
SPLIT = 512
S_BACK = S - SPLIT
D_MODEL = 1024
D_FF = 2048
E = 8
D_FF_E = 512

T_FFN = 256
T_MOE = 256


def _router_body(x_ref, w_ref, d_ref, comb_ref):
    l = jnp.dot(x_ref[...], w_ref[...], preferred_element_type=jnp.float32)
    l = l + d_ref[...]
    iota_e = jax.lax.broadcasted_iota(jnp.int32, l.shape, 1)
    m1 = jnp.max(l, axis=1, keepdims=True)
    a1 = jnp.min(jnp.where(l == m1, iota_e, E), axis=1, keepdims=True)
    masked = jnp.where(iota_e == a1, -jnp.inf, l)
    m2 = jnp.max(masked, axis=1, keepdims=True)
    a2 = jnp.min(jnp.where(masked == m2, iota_e, E), axis=1, keepdims=True)
    g1 = 1.0 / (1.0 + jnp.exp(m1 - m2))
    g0 = 1.0 - g1
    comb = jnp.where(iota_e == a1, g0, 0.0) + jnp.where(iota_e == a2, g1, 0.0)
    comb_ref[...] = comb


def _ffn_body(x_ref, w1_ref, b1_ref, w2_ref, b2_ref, y_ref):
    h = jnp.dot(x_ref[...], w1_ref[...], preferred_element_type=jnp.float32)
    h = jax.nn.gelu(h + b1_ref[...])
    y = jnp.dot(h, w2_ref[...], preferred_element_type=jnp.float32)
    y_ref[...] = y + b2_ref[...]


def _moe_body(x_ref, comb_ref, we1_ref, be1_ref, we2_ref, be2_ref, y_ref,
              out_ref):
    acc = y_ref[...]
    comb = comb_ref[...]
    iota_e = jax.lax.broadcasted_iota(jnp.int32, comb.shape, 1)
    for e in range(E):
        h = jnp.dot(x_ref[...], we1_ref[e],
                    preferred_element_type=jnp.float32)
        h = jax.nn.gelu(h + be1_ref[e])
        o = jnp.dot(h, we2_ref[e], preferred_element_type=jnp.float32)
        o = o + be2_ref[e]
        g = jnp.sum(jnp.where(iota_e == e, comb, 0.0), axis=1, keepdims=True)
        acc = acc + g * o
    out_ref[...] = acc


def kernel(x, id, weight, delta, W1, b1, W2, b2, We1, be1, We2, be2):
    del id
    xf = x.reshape(S, D_MODEL)
    x_back = xf[SPLIT:]

    comb = pl.pallas_call(
        _router_body,
        grid=(S_BACK // T_MOE,),
        in_specs=[
            pl.BlockSpec((T_MOE, D_MODEL), lambda t: (t, 0)),
            pl.BlockSpec((D_MODEL, E), lambda t: (0, 0)),
            pl.BlockSpec((1, E), lambda t: (0, 0)),
        ],
        out_specs=pl.BlockSpec((T_MOE, E), lambda t: (t, 0)),
        out_shape=jax.ShapeDtypeStruct((S_BACK, E), jnp.float32),
    )(x_back, weight, delta.reshape(1, E))

    y = pl.pallas_call(
        _ffn_body,
        grid=(S // T_FFN,),
        in_specs=[
            pl.BlockSpec((T_FFN, D_MODEL), lambda t: (t, 0)),
            pl.BlockSpec((D_MODEL, D_FF), lambda t: (0, 0)),
            pl.BlockSpec((1, D_FF), lambda t: (0, 0)),
            pl.BlockSpec((D_FF, D_MODEL), lambda t: (0, 0)),
            pl.BlockSpec((1, D_MODEL), lambda t: (0, 0)),
        ],
        out_specs=pl.BlockSpec((T_FFN, D_MODEL), lambda t: (t, 0)),
        out_shape=jax.ShapeDtypeStruct((S, D_MODEL), jnp.float32),
    )(xf, W1, b1.reshape(1, D_FF), W2, b2.reshape(1, D_MODEL))

    out_back = pl.pallas_call(
        _moe_body,
        grid=(S_BACK // T_MOE,),
        in_specs=[
            pl.BlockSpec((T_MOE, D_MODEL), lambda t: (t, 0)),
            pl.BlockSpec((T_MOE, E), lambda t: (t, 0)),
            pl.BlockSpec((E, D_MODEL, D_FF_E), lambda t: (0, 0, 0)),
            pl.BlockSpec((E, 1, D_FF_E), lambda t: (0, 0, 0)),
            pl.BlockSpec((E, D_FF_E, D_MODEL), lambda t: (0, 0, 0)),
            pl.BlockSpec((E, 1, D_MODEL), lambda t: (0, 0, 0)),
            pl.BlockSpec((T_MOE, D_MODEL), lambda t: (t, 0)),
        ],
        out_specs=pl.BlockSpec((T_MOE, D_MODEL), lambda t: (t, 0)),
        out_shape=jax.ShapeDtypeStruct((S_BACK, D_MODEL), jnp.float32),
    )(x_back, comb, We1, be1.reshape(E, 1, D_FF_E), We2,
      be2.reshape(E, 1, D_MODEL), y[SPLIT:])

    out = jnp.concatenate([y[:SPLIT], out_back], axis=0)
    return out.reshape(1, S, D_MODEL)
